# baseline (speedup 1.0000x reference)
import functools

import jax
import jax.numpy as jnp
from jax import lax
from jax.experimental import pallas as pl
from jax.experimental.pallas import tpu as pltpu
from jax.experimental.pallas import tpu_sc as plsc

B, L = 8, 2048
NT = B * L
NUM_NODES = 100000
HID = 128
NODE_DIM = 64

NCORES = 2
NSUB = 16
NW = NCORES * NSUB
PASSES = 8
NRANGE = NCORES * PASSES
RNG = NUM_NODES // NRANGE
TPT = NT // NSUB
TPW = NT // NW
CH = 128
NCH = TPT // CH
CAP1 = TPT + CH
NROW2 = CAP1 // CH
MPAD = NT + CH
EPS = 1e-6


def _seg_body(do_cnt, do_e, passes, rng, *refs):
    if do_e:
        m_hbm, idx_hbm, pad_hbm, nv_hbm, zeros_hbm = refs[:5]
        refs = refs[5:]
    else:
        m_hbm, idx_hbm, pad_hbm, zeros_hbm = refs[:4]
        nv_hbm = None
        refs = refs[4:]
    g_hbm = refs[0]
    refs = refs[1:]
    if do_cnt:
        cnt_hbm = refs[0]
        refs = refs[1:]
    if do_e:
        e_hbm = refs[0]
        refs = refs[1:]
    refs = list(refs)
    idx_t = refs.pop(0)
    pos2 = refs.pop(0)
    lid2 = refs.pop(0)
    rowb = refs.pop(0)
    zrow = refs.pop(0)
    if do_e:
        idxe = refs.pop(0)
    if do_cnt:
        padt = refs.pop(0)
        padb = refs.pop(0)
        cntb = refs.pop(0)
        cloc = refs.pop(0)
        cnt_sp = refs.pop(0)
    acc = refs.pop(0)
    semg, semz, semo, semc = refs

    s = lax.axis_index("c")
    t = lax.axis_index("s")
    w = t * NCORES + s

    pltpu.sync_copy(idx_hbm.at[pl.ds(t * TPT, TPT)], idx_t)
    if do_cnt:
        pltpu.sync_copy(pad_hbm.at[pl.ds(t * TPT, TPT)], padt)

        def _zcloc(k, _):
            pltpu.sync_copy(zeros_hbm.at[0], cloc.at[pl.ds(k * CH, CH)])
            return 0
        lax.fori_loop(0, TPT // CH, _zcloc, 0)
    pltpu.sync_copy(zeros_hbm.at[pl.ds((w % 4) * CH, CH)], zrow)

    if do_e:
        pltpu.sync_copy(idx_hbm.at[pl.ds(w * TPW, TPW)], idxe)

        def _shift(k, _):
            idxe[pl.ds(k * 16, 16)] = idxe[pl.ds(k * 16, 16)] >> 1
            return 0
        lax.fori_loop(0, TPW // 16, _shift, 0)

        def _egather(c, _):
            pltpu.async_copy(
                nv_hbm.at[idxe.at[pl.ds(c * CH, CH)]], rowb, semg).wait()
            pltpu.sync_copy(rowb, e_hbm.at[pl.ds(w * TPW + c * CH, CH)])
            return 0
        lax.fori_loop(0, TPW // CH, _egather, 0)

    iota16 = lax.iota(jnp.int32, 16)
    pad_pos = jnp.int32(NT) + w
    dummy_lid = jnp.int32(rng) + t

    def one_pass(p, _):
        lo = (s * passes + p) * rng
        hi = lo + rng

        def prefill(k, _):
            sl = iota16 + k * 16
            plsc.store_scatter(pos2, [sl >> 7, sl & 127],
                               jnp.full((16,), pad_pos, jnp.int32))
            plsc.store_scatter(lid2, [sl >> 7, sl & 127],
                               jnp.full((16,), dummy_lid, jnp.int32))
            return 0
        lax.fori_loop(0, CAP1 // 16, prefill, 0)

        def filt(i, nt):
            v = idx_t[pl.ds(i * 16, 16)]
            inr = (v >= lo) & (v < hi)
            posv = iota16 + (t * TPT + i * 16)
            lidv = v - lo
            pc = plsc.cumsum(inr.astype(jnp.int32))
            slot = jnp.where(inr, nt + pc - 1, jnp.int32(CAP1 - 16) + iota16)
            plsc.store_scatter(pos2, [slot >> 7, slot & 127], posv)
            plsc.store_scatter(lid2, [slot >> 7, slot & 127], lidv)
            return nt + pc[15]
        ntok = lax.fori_loop(0, TPT // 16, filt, jnp.int32(0))
        nch = (ntok + (CH - 1)) // CH

        @pl.when(nch > 0)
        def _():
            pltpu.async_copy(m_hbm.at[pos2.at[0]], rowb, semg)

        def zero_c(c, _):
            pltpu.async_copy(zrow, acc.at[lid2.at[c]], semz).wait()
            if do_cnt:
                pltpu.async_copy(zrow.at[0], cnt_sp.at[lid2.at[c]],
                                 semz).wait()
            return 0
        lax.fori_loop(0, nch, zero_c, 0)
        plsc.subcore_barrier()

        def add_c(c, _):
            @pl.when(c > 0)
            def _():
                pltpu.async_copy(m_hbm.at[pos2.at[c]], rowb, semg)
            pltpu.make_async_copy(m_hbm.at[pos2.at[c]], rowb, semg).wait()
            pltpu.async_copy(rowb, acc.at[lid2.at[c]], semz, add=True).wait()
            if do_cnt:
                for b2 in range(CH // 16):
                    fl = iota16 + (c * CH + b2 * 16)
                    msk = fl < ntok
                    ppos = pos2[c, pl.ds(b2 * 16, 16)] - t * TPT
                    ppos = jnp.where(msk, ppos, 0)
                    pv = plsc.load_gather(padt, [ppos])
                    pv = jnp.where(msk, pv, 0.0)
                    padb[pl.ds(b2 * 16, 16)] = pv
                pltpu.async_copy(padb, cnt_sp.at[lid2.at[c]], semc,
                                 add=True).wait()
            return 0
        lax.fori_loop(0, nch, add_c, 0)
        plsc.subcore_barrier()

        def out_c(c, _):
            pltpu.async_copy(acc.at[lid2.at[c]], rowb, semg).wait()
            pltpu.async_copy(rowb, g_hbm.at[pos2.at[c]], semo)

            @pl.when(c + 1 < nch)
            def _():
                pltpu.make_async_copy(rowb, g_hbm.at[pos2.at[c]], semo).wait()
            if do_cnt:
                pltpu.async_copy(cnt_sp.at[lid2.at[c]], cntb, semc).wait()
                for b2 in range(CH // 16):
                    fl = iota16 + (c * CH + b2 * 16)
                    msk = fl < ntok
                    ppos = pos2[c, pl.ds(b2 * 16, 16)] - t * TPT
                    ppos = jnp.where(msk, ppos, jnp.int32(TPT) + iota16)
                    cv = cntb[pl.ds(b2 * 16, 16)]
                    plsc.store_scatter(cloc, [ppos], cv)
            return 0
        lax.fori_loop(0, nch, out_c, 0)
        plsc.subcore_barrier()

        @pl.when(nch > 0)
        def _():
            pltpu.make_async_copy(rowb, g_hbm.at[pos2.at[0]], semo).wait()
        return 0

    lax.fori_loop(0, passes, one_pass, 0)

    if do_cnt:
        pltpu.sync_copy(cloc.at[pl.ds(0, TPT)],
                        cnt_hbm.at[s, pl.ds(t * TPT, TPT)])


def _make_seg(do_cnt, do_e, passes):
    rng = NUM_NODES // (NCORES * passes)
    mesh = plsc.VectorSubcoreMesh(core_axis_name="c", subcore_axis_name="s")
    out_type = [jax.ShapeDtypeStruct((MPAD, HID), jnp.float32)]
    if do_cnt:
        out_type.append(jax.ShapeDtypeStruct((NCORES, NT), jnp.float32))
    if do_e:
        out_type.append(jax.ShapeDtypeStruct((NT, HID), jnp.float32))
    scratch = [
        pltpu.VMEM((TPT,), jnp.int32),
        pltpu.VMEM((NROW2, CH), jnp.int32),
        pltpu.VMEM((NROW2, CH), jnp.int32),
        pltpu.VMEM((CH, HID), jnp.float32),
        pltpu.VMEM((CH, HID), jnp.float32),
    ]
    if do_e:
        scratch += [
            pltpu.VMEM((TPW,), jnp.int32),
        ]
    if do_cnt:
        scratch += [
            pltpu.VMEM((TPT,), jnp.float32),
            pltpu.VMEM((CH,), jnp.float32),
            pltpu.VMEM((CH,), jnp.float32),
            pltpu.VMEM((TPT + 16,), jnp.float32),
            pltpu.VMEM_SHARED((rng + NSUB,), jnp.float32),
        ]
    scratch += [
        pltpu.VMEM_SHARED((rng + NSUB, HID), jnp.float32),
        pltpu.SemaphoreType.DMA,
        pltpu.SemaphoreType.DMA,
        pltpu.SemaphoreType.DMA,
        pltpu.SemaphoreType.DMA,
    ]
    return pl.kernel(
        functools.partial(_seg_body, do_cnt, do_e, passes, rng),
        out_type=tuple(out_type) if len(out_type) > 1 else out_type[0],
        mesh=mesh,
        scratch_types=scratch,
        compiler_params=pltpu.CompilerParams(needs_layout_passes=False,
                                             use_tc_tiling_on_sc=True),
    )


def _aux_body(*refs):
    (idx_hbm, pad_hbm, nv0_hbm, nv1_hbm, zeros_hbm,
     cnt_hbm, e0_hbm, e1_hbm,
     idx_t, idxe, pos2, lid2, rowb, zvec, padt, padb, cntb, cloc,
     cnt_sp, semg, semz, semc) = refs
    s = lax.axis_index("c")
    t = lax.axis_index("s")
    w = t * NCORES + s
    rng = NUM_NODES // NCORES

    pltpu.sync_copy(idx_hbm.at[pl.ds(t * TPT, TPT)], idx_t)
    pltpu.sync_copy(pad_hbm.at[pl.ds(t * TPT, TPT)], padt)
    pltpu.sync_copy(zeros_hbm.at[(w % 4) * CH], zvec)

    def _zcloc(k, _):
        pltpu.sync_copy(zeros_hbm.at[(w % 4) * CH], cloc.at[pl.ds(k * CH, CH)])
        return 0
    lax.fori_loop(0, TPT // CH, _zcloc, 0)

    pltpu.sync_copy(idx_hbm.at[pl.ds(w * TPW, TPW)], idxe)

    def _shift(k, _):
        idxe[pl.ds(k * 16, 16)] = idxe[pl.ds(k * 16, 16)] >> 1
        return 0
    lax.fori_loop(0, TPW // 16, _shift, 0)

    def _eg0(c, _):
        pltpu.async_copy(nv0_hbm.at[idxe.at[pl.ds(c * CH, CH)]], rowb,
                         semg).wait()
        pltpu.sync_copy(rowb, e0_hbm.at[pl.ds(w * TPW + c * CH, CH)])
        return 0
    lax.fori_loop(0, TPW // CH, _eg0, 0)

    def _eg1(c, _):
        pltpu.async_copy(nv1_hbm.at[idxe.at[pl.ds(c * CH, CH)]], rowb,
                         semg).wait()
        pltpu.sync_copy(rowb, e1_hbm.at[pl.ds(w * TPW + c * CH, CH)])
        return 0
    lax.fori_loop(0, TPW // CH, _eg1, 0)

    iota16 = lax.iota(jnp.int32, 16)
    pad_pos = jnp.int32(NT) + w
    dummy_lid = jnp.int32(rng) + t
    lo = s * rng
    hi = lo + rng

    def prefill(k, _):
        sl = iota16 + k * 16
        plsc.store_scatter(pos2, [sl >> 7, sl & 127],
                           jnp.full((16,), pad_pos, jnp.int32))
        plsc.store_scatter(lid2, [sl >> 7, sl & 127],
                           jnp.full((16,), dummy_lid, jnp.int32))
        return 0
    lax.fori_loop(0, CAP1 // 16, prefill, 0)

    def filt(i, nt):
        v = idx_t[pl.ds(i * 16, 16)]
        inr = (v >= lo) & (v < hi)
        posv = iota16 + (t * TPT + i * 16)
        lidv = v - lo
        pc = plsc.cumsum(inr.astype(jnp.int32))
        slot = jnp.where(inr, nt + pc - 1, jnp.int32(CAP1 - 16) + iota16)
        plsc.store_scatter(pos2, [slot >> 7, slot & 127], posv)
        plsc.store_scatter(lid2, [slot >> 7, slot & 127], lidv)
        return nt + pc[15]
    ntok = lax.fori_loop(0, TPT // 16, filt, jnp.int32(0))
    nch = (ntok + (CH - 1)) // CH

    def zero_c(c, _):
        pltpu.async_copy(zvec, cnt_sp.at[lid2.at[c]], semz).wait()
        return 0
    lax.fori_loop(0, nch, zero_c, 0)
    plsc.subcore_barrier()

    def add_c(c, _):
        for b2 in range(CH // 16):
            fl = iota16 + (c * CH + b2 * 16)
            msk = fl < ntok
            ppos = pos2[c, pl.ds(b2 * 16, 16)] - t * TPT
            ppos = jnp.where(msk, ppos, 0)
            pv = plsc.load_gather(padt, [ppos])
            pv = jnp.where(msk, pv, 0.0)
            padb[pl.ds(b2 * 16, 16)] = pv
        pltpu.async_copy(padb, cnt_sp.at[lid2.at[c]], semc, add=True).wait()
        return 0
    lax.fori_loop(0, nch, add_c, 0)
    plsc.subcore_barrier()

    def out_c(c, _):
        pltpu.async_copy(cnt_sp.at[lid2.at[c]], cntb, semc).wait()
        for b2 in range(CH // 16):
            fl = iota16 + (c * CH + b2 * 16)
            msk = fl < ntok
            ppos = pos2[c, pl.ds(b2 * 16, 16)] - t * TPT
            ppos = jnp.where(msk, ppos, jnp.int32(TPT) + iota16)
            cv = cntb[pl.ds(b2 * 16, 16)]
            plsc.store_scatter(cloc, [ppos], cv)
        return 0
    lax.fori_loop(0, nch, out_c, 0)

    pltpu.sync_copy(cloc.at[pl.ds(0, TPT)],
                    cnt_hbm.at[s, pl.ds(t * TPT, TPT)])


def _make_aux():
    rng = NUM_NODES // NCORES
    mesh = plsc.VectorSubcoreMesh(core_axis_name="c", subcore_axis_name="s")
    return pl.kernel(
        _aux_body,
        out_type=(jax.ShapeDtypeStruct((NCORES, NT), jnp.float32),
                  jax.ShapeDtypeStruct((NT, HID), jnp.float32),
                  jax.ShapeDtypeStruct((NT, HID), jnp.float32)),
        mesh=mesh,
        scratch_types=[
            pltpu.VMEM((TPT,), jnp.int32),
            pltpu.VMEM((TPW,), jnp.int32),
            pltpu.VMEM((NROW2, CH), jnp.int32),
            pltpu.VMEM((NROW2, CH), jnp.int32),
            pltpu.VMEM((CH, HID), jnp.float32),
            pltpu.VMEM((CH,), jnp.float32),
            pltpu.VMEM((TPT,), jnp.float32),
            pltpu.VMEM((CH,), jnp.float32),
            pltpu.VMEM((CH,), jnp.float32),
            pltpu.VMEM((TPT + 16,), jnp.float32),
            pltpu.VMEM_SHARED((rng + NSUB,), jnp.float32),
            pltpu.SemaphoreType.DMA,
            pltpu.SemaphoreType.DMA,
            pltpu.SemaphoreType.DMA,
        ],
        compiler_params=pltpu.CompilerParams(needs_layout_passes=False,
                                             use_tc_tiling_on_sc=True),
    )


GRID = 8
BLK = NT // GRID


def _row_spec(width):
    return pl.BlockSpec((BLK, width), lambda i: (i, 0))


def _skinny_spec():
    return pl.BlockSpec((BLK // 128, 128), lambda i: (i, 0))


def _full_spec(shape):
    return pl.BlockSpec(shape, lambda i: (0,) * len(shape))


def _col(x):
    i0 = lax.broadcasted_iota(jnp.int32, (128, 128), 0)
    i1 = lax.broadcasted_iota(jnp.int32, (128, 128), 1)
    ident = (i0 == i1).astype(jnp.float32)
    x = x.astype(jnp.float32)
    d = jnp.concatenate([ident * x[r:r + 1, :] for r in range(BLK // 128)],
                        axis=0)
    ones = jnp.ones((128, HID), jnp.float32)
    return jnp.dot(d, ones, preferred_element_type=jnp.float32)


def _h0_temb(val, ts, predm, padm, w_enc, b_enc, ptok, ifreq):
    x = val * w_enc + b_enc
    ang = ts[:, :HID // 2] * ifreq
    temb = jnp.concatenate([jnp.sin(ang), jnp.cos(ang)], axis=1) * padm
    h0 = (x + temb) * padm * (1.0 - predm) + ptok * predm
    return h0, temb


def _tc1_body(val_r, ts_r, predm_r, padm_r, wenc_r, benc_r, ptok_r, ifreq_r,
              wm_r, m_r):
    h0, _ = _h0_temb(_col(val_r[...]), _col(ts_r[...]), _col(predm_r[...]),
                     _col(padm_r[...]), wenc_r[...], benc_r[...], ptok_r[...],
                     ifreq_r[...])
    m_r[...] = jnp.dot(h0 * _col(padm_r[...]), wm_r[...],
                       preferred_element_type=jnp.float32)


def _tc1(val, ts, predm, padm, wenc, benc, ptok, ifreq, wm):
    return pl.pallas_call(
        _tc1_body,
        grid=(GRID,),
        in_specs=[_skinny_spec(), _skinny_spec(), _skinny_spec(),
                  _skinny_spec(),
                  _full_spec((1, HID)), _full_spec((1, HID)),
                  _full_spec((1, HID)), _full_spec((1, HID // 2)),
                  _full_spec((HID, HID))],
        out_specs=_row_spec(HID),
        out_shape=jax.ShapeDtypeStruct((MPAD, HID), jnp.float32),
    )(val, ts, predm, padm, wenc, benc, ptok, ifreq, wm)


def _tc2_body(g_r, cnta_r, cntb_r, padm_r, wm_r, m_r):
    inv = 1.0 / (_col(cnta_r[...]) + _col(cntb_r[...]) + EPS)
    hk = g_r[...] * inv
    m_r[...] = jnp.dot(hk * _col(padm_r[...]), wm_r[...],
                       preferred_element_type=jnp.float32)


def _tc2(g, cnta, cntb, padm, wm):
    return pl.pallas_call(
        _tc2_body,
        grid=(GRID,),
        in_specs=[_row_spec(HID), _skinny_spec(), _skinny_spec(),
                  _skinny_spec(), _full_spec((HID, HID))],
        out_specs=_row_spec(HID),
        out_shape=jax.ShapeDtypeStruct((MPAD, HID), jnp.float32),
    )(g, cnta, cntb, padm, wm)


def _layer_tail(h_in, g, inv, e2, par, temb, wn, wo, bo, gamma, beta):
    hk = g * inv
    left = e2[:, :NODE_DIM]
    right = e2[:, NODE_DIM:]
    e = left + (right - left) * par[:, :NODE_DIM]
    pre = h_in + hk + jnp.dot(e, wn, preferred_element_type=jnp.float32) + temb
    h_new = jax.nn.relu(
        jnp.dot(pre, wo, preferred_element_type=jnp.float32) + bo)
    x = h_in + h_new
    mu = jnp.mean(x, axis=-1, keepdims=True)
    var = jnp.mean((x - mu) ** 2, axis=-1, keepdims=True)
    return (x - mu) / jnp.sqrt(var + 1e-5) * gamma + beta


def _tc3_body(val_r, ts_r, predm_r, padm_r, g_r, cnta_r, cntb_r, e_r, idx_r,
              wenc_r, benc_r, ptok_r, ifreq_r,
              wn_r, wo_r, bo_r, gam_r, bet_r, wm_r, h1_r, m_r):
    h0, temb = _h0_temb(_col(val_r[...]), _col(ts_r[...]), _col(predm_r[...]),
                        _col(padm_r[...]), wenc_r[...], benc_r[...],
                        ptok_r[...], ifreq_r[...])
    inv = 1.0 / (_col(cnta_r[...]) + _col(cntb_r[...]) + EPS)
    par = _col((idx_r[...] & 1).astype(jnp.float32))
    h1 = _layer_tail(h0, g_r[...], inv, e_r[...], par, temb,
                     wn_r[...], wo_r[...], bo_r[...], gam_r[...], bet_r[...])
    h1_r[...] = h1
    m_r[...] = jnp.dot(h1 * _col(padm_r[...]), wm_r[...],
                       preferred_element_type=jnp.float32)


def _tc3(val, ts, predm, padm, g, cnta, cntb, e, idx2, wenc, benc, ptok,
         ifreq, wn, wo, bo, gam, bet, wm):
    return pl.pallas_call(
        _tc3_body,
        grid=(GRID,),
        in_specs=[_skinny_spec(), _skinny_spec(), _skinny_spec(),
                  _skinny_spec(),
                  _row_spec(HID), _skinny_spec(), _skinny_spec(),
                  _row_spec(HID), _skinny_spec(),
                  _full_spec((1, HID)), _full_spec((1, HID)),
                  _full_spec((1, HID)), _full_spec((1, HID // 2)),
                  _full_spec((NODE_DIM, HID)), _full_spec((HID, HID)),
                  _full_spec((1, HID)), _full_spec((1, HID)),
                  _full_spec((1, HID)), _full_spec((HID, HID))],
        out_specs=(_row_spec(HID), _row_spec(HID)),
        out_shape=(jax.ShapeDtypeStruct((NT, HID), jnp.float32),
                   jax.ShapeDtypeStruct((MPAD, HID), jnp.float32)),
    )(val, ts, predm, padm, g, cnta, cntb, e, idx2, wenc, benc, ptok, ifreq,
      wn, wo, bo, gam, bet, wm)


def _tc4_body(val_r, ts_r, predm_r, padm_r, h1_r, g_r, cnta_r, cntb_r, e_r,
              idx_r, wenc_r, benc_r, ptok_r, ifreq_r,
              wn_r, wo_r, bo_r, gam_r, bet_r, lam_r, wp_r, bp_r, out_r):
    h0, temb = _h0_temb(_col(val_r[...]), _col(ts_r[...]), _col(predm_r[...]),
                        _col(padm_r[...]), wenc_r[...], benc_r[...],
                        ptok_r[...], ifreq_r[...])
    inv = 1.0 / (_col(cnta_r[...]) + _col(cntb_r[...]) + EPS)
    par = _col((idx_r[...] & 1).astype(jnp.float32))
    h2 = _layer_tail(h1_r[...], g_r[...], inv, e_r[...], par, temb,
                     wn_r[...], wo_r[...], bo_r[...], gam_r[...], bet_r[...])
    lam = lam_r[0, 0]
    h = h2 * lam + h0 * (1.0 - lam)
    out = jnp.dot(h, wp_r[...], preferred_element_type=jnp.float32) \
        + bp_r[...]
    rows = [jnp.swapaxes(out[r * 128:(r + 1) * 128, :], 0, 1)
            for r in range(BLK // 128)]
    out_r[...] = jnp.concatenate(rows, axis=0)


def _tc4(val, ts, predm, padm, h1, g, cnta, cntb, e, idx2, wenc, benc, ptok,
         ifreq, wn, wo, bo, gam, bet, lam, wp, bp):
    return pl.pallas_call(
        _tc4_body,
        grid=(GRID,),
        in_specs=[_skinny_spec(), _skinny_spec(), _skinny_spec(),
                  _skinny_spec(),
                  _row_spec(HID), _row_spec(HID), _skinny_spec(),
                  _skinny_spec(), _row_spec(HID), _skinny_spec(),
                  _full_spec((1, HID)), _full_spec((1, HID)),
                  _full_spec((1, HID)), _full_spec((1, HID // 2)),
                  _full_spec((NODE_DIM, HID)), _full_spec((HID, HID)),
                  _full_spec((1, HID)), _full_spec((1, HID)),
                  _full_spec((1, HID)), _full_spec((1, 1)),
                  _full_spec((HID, 1)), _full_spec((1, 1))],
        out_specs=_skinny_spec(),
        out_shape=jax.ShapeDtypeStruct((128, 128), jnp.float32),
    )(val, ts, predm, padm, h1, g, cnta, cntb, e, idx2, wenc, benc, ptok,
      ifreq, wn, wo, bo, gam, bet, lam, wp, bp)


def kernel(batch_value, batch_timestamp, batch_var_idx, batch_pred_mask,
           batch_pad_mask, W_enc, b_enc, pred_token, nodevec_0, W_node_0,
           W_msg_0, W_out_0, b_out_0, gamma_0, beta_0, nodevec_1, W_node_1,
           W_msg_1, W_out_1, b_out_1, gamma_1, beta_1, lambda_, W_proj,
           b_proj):
    val = batch_value.reshape(128, 128)
    ts = batch_timestamp.reshape(128, 128)
    predm = batch_pred_mask.reshape(128, 128)
    padm = batch_pad_mask.reshape(128, 128)
    idx = batch_var_idx.reshape(NT).astype(jnp.int32)
    idx2 = idx.reshape(128, 128)
    nv0 = nodevec_0.reshape(NUM_NODES // 2, 2 * NODE_DIM)
    nv1 = nodevec_1.reshape(NUM_NODES // 2, 2 * NODE_DIM)
    pad_flat = batch_pad_mask.reshape(NT)
    wenc = W_enc.reshape(1, HID)
    benc = b_enc.reshape(1, HID)
    ptok = pred_token.reshape(1, HID)
    half = HID // 2
    ifreq = (1.0 / (10000.0 ** (jnp.arange(half, dtype=jnp.float32) / half))
             ).reshape(1, half)
    bo0 = b_out_0.reshape(1, HID)
    bo1 = b_out_1.reshape(1, HID)
    gam0 = gamma_0.reshape(1, HID)
    bet0 = beta_0.reshape(1, HID)
    gam1 = gamma_1.reshape(1, HID)
    bet1 = beta_1.reshape(1, HID)
    lam = jnp.asarray(lambda_, jnp.float32).reshape(1, 1)
    bp = b_proj.reshape(1, 1)
    zeros = jnp.zeros((4 * CH, HID), jnp.float32)

    aux = _make_aux()
    seg = _make_seg(False, False, 5)

    cnt2, e0, e1 = aux(idx, pad_flat, nv0, nv1, zeros)
    cnta = cnt2[0].reshape(128, 128)
    cntb = cnt2[1].reshape(128, 128)
    m0 = _tc1(val, ts, predm, padm, wenc, benc, ptok, ifreq, W_msg_0)
    g0 = seg(m0, idx, pad_flat, zeros)
    m1 = _tc2(g0, cnta, cntb, padm, W_msg_0)
    g1 = seg(m1, idx, pad_flat, zeros)
    h1, m2 = _tc3(val, ts, predm, padm, g1, cnta, cntb, e0, idx2, wenc, benc,
                  ptok, ifreq, W_node_0, W_out_0, bo0, gam0, bet0, W_msg_1)
    g2 = seg(m2, idx, pad_flat, zeros)
    m3 = _tc2(g2, cnta, cntb, padm, W_msg_1)
    g3 = seg(m3, idx, pad_flat, zeros)
    out = _tc4(val, ts, predm, padm, h1, g3, cnta, cntb, e1, idx2, wenc,
               benc, ptok, ifreq, W_node_1, W_out_1, bo1, gam1, bet1, lam,
               W_proj, bp)
    return out.reshape(B, L, 1)


# speedup vs baseline: 1.0481x; 1.0481x over previous
import functools

import jax
import jax.numpy as jnp
from jax import lax
from jax.experimental import pallas as pl
from jax.experimental.pallas import tpu as pltpu
from jax.experimental.pallas import tpu_sc as plsc

B, L = 8, 2048
NT = B * L
NUM_NODES = 100000
HID = 128
NODE_DIM = 64

NCORES = 2
NSUB = 16
NW = NCORES * NSUB
PASSES = 8
NRANGE = NCORES * PASSES
RNG = NUM_NODES // NRANGE
TPT = NT // NSUB
TPW = NT // NW
CH = 128
NCH = TPT // CH
CAP1 = TPT + CH
NROW2 = CAP1 // CH
MPAD = NT + CH
EPS = 1e-6


def _seg_body(do_cnt, do_e, passes, rng, *refs):
    if do_e:
        m_hbm, idx_hbm, pad_hbm, nv_hbm, zeros_hbm = refs[:5]
        refs = refs[5:]
    else:
        m_hbm, idx_hbm, pad_hbm, zeros_hbm = refs[:4]
        nv_hbm = None
        refs = refs[4:]
    g_hbm = refs[0]
    refs = refs[1:]
    if do_cnt:
        cnt_hbm = refs[0]
        refs = refs[1:]
    if do_e:
        e_hbm = refs[0]
        refs = refs[1:]
    refs = list(refs)
    idx_t = refs.pop(0)
    pos2 = refs.pop(0)
    lid2 = refs.pop(0)
    rowb = refs.pop(0)
    zrow = refs.pop(0)
    if do_e:
        idxe = refs.pop(0)
    if do_cnt:
        padt = refs.pop(0)
        padb = refs.pop(0)
        cntb = refs.pop(0)
        cloc = refs.pop(0)
        cnt_sp = refs.pop(0)
    acc = refs.pop(0)
    semg, semz, semo, semc = refs

    s = lax.axis_index("c")
    t = lax.axis_index("s")
    w = t * NCORES + s

    pltpu.sync_copy(idx_hbm.at[pl.ds(t * TPT, TPT)], idx_t)
    if do_cnt:
        pltpu.sync_copy(pad_hbm.at[pl.ds(t * TPT, TPT)], padt)

        def _zcloc(k, _):
            pltpu.sync_copy(zeros_hbm.at[0], cloc.at[pl.ds(k * CH, CH)])
            return 0
        lax.fori_loop(0, TPT // CH, _zcloc, 0)
    pltpu.sync_copy(zeros_hbm.at[pl.ds((w % 4) * CH, CH)], zrow)

    if do_e:
        pltpu.sync_copy(idx_hbm.at[pl.ds(w * TPW, TPW)], idxe)

        def _shift(k, _):
            idxe[pl.ds(k * 16, 16)] = idxe[pl.ds(k * 16, 16)] >> 1
            return 0
        lax.fori_loop(0, TPW // 16, _shift, 0)

        def _egather(c, _):
            pltpu.async_copy(
                nv_hbm.at[idxe.at[pl.ds(c * CH, CH)]], rowb, semg).wait()
            pltpu.sync_copy(rowb, e_hbm.at[pl.ds(w * TPW + c * CH, CH)])
            return 0
        lax.fori_loop(0, TPW // CH, _egather, 0)

    iota16 = lax.iota(jnp.int32, 16)
    pad_pos = jnp.int32(NT) + w
    dummy_lid = jnp.int32(rng) + t

    def one_pass(p, _):
        lo = (s * passes + p) * rng
        hi = lo + rng

        def prefill(k, _):
            sl = iota16 + k * 16
            plsc.store_scatter(pos2, [sl >> 7, sl & 127],
                               jnp.full((16,), pad_pos, jnp.int32))
            plsc.store_scatter(lid2, [sl >> 7, sl & 127],
                               jnp.full((16,), dummy_lid, jnp.int32))
            return 0
        lax.fori_loop(0, CAP1 // 16, prefill, 0)

        def filt(i, nt):
            v = idx_t[pl.ds(i * 16, 16)]
            inr = (v >= lo) & (v < hi)
            posv = iota16 + (t * TPT + i * 16)
            lidv = v - lo
            pc = plsc.cumsum(inr.astype(jnp.int32))
            slot = jnp.where(inr, nt + pc - 1, jnp.int32(CAP1 - 16) + iota16)
            plsc.store_scatter(pos2, [slot >> 7, slot & 127], posv)
            plsc.store_scatter(lid2, [slot >> 7, slot & 127], lidv)
            return nt + pc[15]
        ntok = lax.fori_loop(0, TPT // 16, filt, jnp.int32(0))
        nch = (ntok + (CH - 1)) // CH

        @pl.when(nch > 0)
        def _():
            pltpu.async_copy(m_hbm.at[pos2.at[0]], rowb, semg)

        def zero_c(c, _):
            pltpu.async_copy(zrow, acc.at[lid2.at[c]], semz).wait()
            if do_cnt:
                pltpu.async_copy(zrow.at[0], cnt_sp.at[lid2.at[c]],
                                 semz).wait()
            return 0
        lax.fori_loop(0, nch, zero_c, 0)
        plsc.subcore_barrier()

        def add_c(c, _):
            @pl.when(c > 0)
            def _():
                pltpu.async_copy(m_hbm.at[pos2.at[c]], rowb, semg)
            pltpu.make_async_copy(m_hbm.at[pos2.at[c]], rowb, semg).wait()
            pltpu.async_copy(rowb, acc.at[lid2.at[c]], semz, add=True).wait()
            if do_cnt:
                for b2 in range(CH // 16):
                    fl = iota16 + (c * CH + b2 * 16)
                    msk = fl < ntok
                    ppos = pos2[c, pl.ds(b2 * 16, 16)] - t * TPT
                    ppos = jnp.where(msk, ppos, 0)
                    pv = plsc.load_gather(padt, [ppos])
                    pv = jnp.where(msk, pv, 0.0)
                    padb[pl.ds(b2 * 16, 16)] = pv
                pltpu.async_copy(padb, cnt_sp.at[lid2.at[c]], semc,
                                 add=True).wait()
            return 0
        lax.fori_loop(0, nch, add_c, 0)
        plsc.subcore_barrier()

        def out_c(c, _):
            pltpu.async_copy(acc.at[lid2.at[c]], rowb, semg).wait()
            pltpu.async_copy(rowb, g_hbm.at[pos2.at[c]], semo)

            @pl.when(c + 1 < nch)
            def _():
                pltpu.make_async_copy(rowb, g_hbm.at[pos2.at[c]], semo).wait()
            if do_cnt:
                pltpu.async_copy(cnt_sp.at[lid2.at[c]], cntb, semc).wait()
                for b2 in range(CH // 16):
                    fl = iota16 + (c * CH + b2 * 16)
                    msk = fl < ntok
                    ppos = pos2[c, pl.ds(b2 * 16, 16)] - t * TPT
                    ppos = jnp.where(msk, ppos, jnp.int32(TPT) + iota16)
                    cv = cntb[pl.ds(b2 * 16, 16)]
                    plsc.store_scatter(cloc, [ppos], cv)
            return 0
        lax.fori_loop(0, nch, out_c, 0)
        plsc.subcore_barrier()

        @pl.when(nch > 0)
        def _():
            pltpu.make_async_copy(rowb, g_hbm.at[pos2.at[0]], semo).wait()
        return 0

    lax.fori_loop(0, passes, one_pass, 0)

    if do_cnt:
        pltpu.sync_copy(cloc.at[pl.ds(0, TPT)],
                        cnt_hbm.at[s, pl.ds(t * TPT, TPT)])


def _make_seg(do_cnt, do_e, passes):
    rng = NUM_NODES // (NCORES * passes)
    mesh = plsc.VectorSubcoreMesh(core_axis_name="c", subcore_axis_name="s")
    out_type = [jax.ShapeDtypeStruct((MPAD, HID), jnp.float32)]
    if do_cnt:
        out_type.append(jax.ShapeDtypeStruct((NCORES, NT), jnp.float32))
    if do_e:
        out_type.append(jax.ShapeDtypeStruct((NT, HID), jnp.float32))
    scratch = [
        pltpu.VMEM((TPT,), jnp.int32),
        pltpu.VMEM((NROW2, CH), jnp.int32),
        pltpu.VMEM((NROW2, CH), jnp.int32),
        pltpu.VMEM((CH, HID), jnp.float32),
        pltpu.VMEM((CH, HID), jnp.float32),
    ]
    if do_e:
        scratch += [
            pltpu.VMEM((TPW,), jnp.int32),
        ]
    if do_cnt:
        scratch += [
            pltpu.VMEM((TPT,), jnp.float32),
            pltpu.VMEM((CH,), jnp.float32),
            pltpu.VMEM((CH,), jnp.float32),
            pltpu.VMEM((TPT + 16,), jnp.float32),
            pltpu.VMEM_SHARED((rng + NSUB,), jnp.float32),
        ]
    scratch += [
        pltpu.VMEM_SHARED((rng + NSUB, HID), jnp.float32),
        pltpu.SemaphoreType.DMA,
        pltpu.SemaphoreType.DMA,
        pltpu.SemaphoreType.DMA,
        pltpu.SemaphoreType.DMA,
    ]
    return pl.kernel(
        functools.partial(_seg_body, do_cnt, do_e, passes, rng),
        out_type=tuple(out_type) if len(out_type) > 1 else out_type[0],
        mesh=mesh,
        scratch_types=scratch,
        compiler_params=pltpu.CompilerParams(needs_layout_passes=False,
                                             use_tc_tiling_on_sc=True),
    )


GRID = 8
BLK = NT // GRID


def _row_spec(width):
    return pl.BlockSpec((BLK, width), lambda i: (i, 0))


def _skinny_spec():
    return pl.BlockSpec((BLK // 128, 128), lambda i: (i, 0))


def _full_spec(shape):
    return pl.BlockSpec(shape, lambda i: (0,) * len(shape))


def _col(x):
    i0 = lax.broadcasted_iota(jnp.int32, (128, 128), 0)
    i1 = lax.broadcasted_iota(jnp.int32, (128, 128), 1)
    ident = (i0 == i1).astype(jnp.float32)
    x = x.astype(jnp.float32)
    d = jnp.concatenate([ident * x[r:r + 1, :] for r in range(BLK // 128)],
                        axis=0)
    ones = jnp.ones((128, HID), jnp.float32)
    return jnp.dot(d, ones, preferred_element_type=jnp.float32)


def _h0_temb(val, ts, predm, padm, w_enc, b_enc, ptok, ifreq):
    x = val * w_enc + b_enc
    ang = ts[:, :HID // 2] * ifreq
    temb = jnp.concatenate([jnp.sin(ang), jnp.cos(ang)], axis=1) * padm
    h0 = (x + temb) * padm * (1.0 - predm) + ptok * predm
    return h0, temb


def _tc1_body(val_r, ts_r, predm_r, padm_r, wenc_r, benc_r, ptok_r, ifreq_r,
              wm_r, m_r):
    h0, _ = _h0_temb(_col(val_r[...]), _col(ts_r[...]), _col(predm_r[...]),
                     _col(padm_r[...]), wenc_r[...], benc_r[...], ptok_r[...],
                     ifreq_r[...])
    m_r[...] = jnp.dot(h0 * _col(padm_r[...]), wm_r[...],
                       preferred_element_type=jnp.float32)


def _tc1(val, ts, predm, padm, wenc, benc, ptok, ifreq, wm):
    return pl.pallas_call(
        _tc1_body,
        grid=(GRID,),
        in_specs=[_skinny_spec(), _skinny_spec(), _skinny_spec(),
                  _skinny_spec(),
                  _full_spec((1, HID)), _full_spec((1, HID)),
                  _full_spec((1, HID)), _full_spec((1, HID // 2)),
                  _full_spec((HID, HID))],
        out_specs=_row_spec(HID),
        out_shape=jax.ShapeDtypeStruct((MPAD, HID), jnp.float32),
    )(val, ts, predm, padm, wenc, benc, ptok, ifreq, wm)


def _tc2_body(g_r, cnta_r, cntb_r, padm_r, wm_r, m_r):
    inv = 1.0 / (_col(cnta_r[...]) + _col(cntb_r[...]) + EPS)
    hk = g_r[...] * inv
    m_r[...] = jnp.dot(hk * _col(padm_r[...]), wm_r[...],
                       preferred_element_type=jnp.float32)


def _tc2(g, cnta, cntb, padm, wm):
    return pl.pallas_call(
        _tc2_body,
        grid=(GRID,),
        in_specs=[_row_spec(HID), _skinny_spec(), _skinny_spec(),
                  _skinny_spec(), _full_spec((HID, HID))],
        out_specs=_row_spec(HID),
        out_shape=jax.ShapeDtypeStruct((MPAD, HID), jnp.float32),
    )(g, cnta, cntb, padm, wm)


def _layer_tail(h_in, g, inv, e2, par, temb, wn, wo, bo, gamma, beta):
    hk = g * inv
    left = e2[:, :NODE_DIM]
    right = e2[:, NODE_DIM:]
    e = left + (right - left) * par[:, :NODE_DIM]
    pre = h_in + hk + jnp.dot(e, wn, preferred_element_type=jnp.float32) + temb
    h_new = jax.nn.relu(
        jnp.dot(pre, wo, preferred_element_type=jnp.float32) + bo)
    x = h_in + h_new
    mu = jnp.mean(x, axis=-1, keepdims=True)
    var = jnp.mean((x - mu) ** 2, axis=-1, keepdims=True)
    return (x - mu) / jnp.sqrt(var + 1e-5) * gamma + beta


def _tc3_body(val_r, ts_r, predm_r, padm_r, g_r, cnta_r, cntb_r, e_r, idx_r,
              wenc_r, benc_r, ptok_r, ifreq_r,
              wn_r, wo_r, bo_r, gam_r, bet_r, wm_r, h1_r, m_r):
    h0, temb = _h0_temb(_col(val_r[...]), _col(ts_r[...]), _col(predm_r[...]),
                        _col(padm_r[...]), wenc_r[...], benc_r[...],
                        ptok_r[...], ifreq_r[...])
    inv = 1.0 / (_col(cnta_r[...]) + _col(cntb_r[...]) + EPS)
    par = _col((idx_r[...] & 1).astype(jnp.float32))
    h1 = _layer_tail(h0, g_r[...], inv, e_r[...], par, temb,
                     wn_r[...], wo_r[...], bo_r[...], gam_r[...], bet_r[...])
    h1_r[...] = h1
    m_r[...] = jnp.dot(h1 * _col(padm_r[...]), wm_r[...],
                       preferred_element_type=jnp.float32)


def _tc3(val, ts, predm, padm, g, cnta, cntb, e, idx2, wenc, benc, ptok,
         ifreq, wn, wo, bo, gam, bet, wm):
    return pl.pallas_call(
        _tc3_body,
        grid=(GRID,),
        in_specs=[_skinny_spec(), _skinny_spec(), _skinny_spec(),
                  _skinny_spec(),
                  _row_spec(HID), _skinny_spec(), _skinny_spec(),
                  _row_spec(HID), _skinny_spec(),
                  _full_spec((1, HID)), _full_spec((1, HID)),
                  _full_spec((1, HID)), _full_spec((1, HID // 2)),
                  _full_spec((NODE_DIM, HID)), _full_spec((HID, HID)),
                  _full_spec((1, HID)), _full_spec((1, HID)),
                  _full_spec((1, HID)), _full_spec((HID, HID))],
        out_specs=(_row_spec(HID), _row_spec(HID)),
        out_shape=(jax.ShapeDtypeStruct((NT, HID), jnp.float32),
                   jax.ShapeDtypeStruct((MPAD, HID), jnp.float32)),
    )(val, ts, predm, padm, g, cnta, cntb, e, idx2, wenc, benc, ptok, ifreq,
      wn, wo, bo, gam, bet, wm)


def _tc4_body(val_r, ts_r, predm_r, padm_r, h1_r, g_r, cnta_r, cntb_r, e_r,
              idx_r, wenc_r, benc_r, ptok_r, ifreq_r,
              wn_r, wo_r, bo_r, gam_r, bet_r, lam_r, wp_r, bp_r, out_r):
    h0, temb = _h0_temb(_col(val_r[...]), _col(ts_r[...]), _col(predm_r[...]),
                        _col(padm_r[...]), wenc_r[...], benc_r[...],
                        ptok_r[...], ifreq_r[...])
    inv = 1.0 / (_col(cnta_r[...]) + _col(cntb_r[...]) + EPS)
    par = _col((idx_r[...] & 1).astype(jnp.float32))
    h2 = _layer_tail(h1_r[...], g_r[...], inv, e_r[...], par, temb,
                     wn_r[...], wo_r[...], bo_r[...], gam_r[...], bet_r[...])
    lam = lam_r[0, 0]
    h = h2 * lam + h0 * (1.0 - lam)
    out = jnp.dot(h, wp_r[...], preferred_element_type=jnp.float32) \
        + bp_r[...]
    rows = [jnp.swapaxes(out[r * 128:(r + 1) * 128, :], 0, 1)
            for r in range(BLK // 128)]
    out_r[...] = jnp.concatenate(rows, axis=0)


def _tc4(val, ts, predm, padm, h1, g, cnta, cntb, e, idx2, wenc, benc, ptok,
         ifreq, wn, wo, bo, gam, bet, lam, wp, bp):
    return pl.pallas_call(
        _tc4_body,
        grid=(GRID,),
        in_specs=[_skinny_spec(), _skinny_spec(), _skinny_spec(),
                  _skinny_spec(),
                  _row_spec(HID), _row_spec(HID), _skinny_spec(),
                  _skinny_spec(), _row_spec(HID), _skinny_spec(),
                  _full_spec((1, HID)), _full_spec((1, HID)),
                  _full_spec((1, HID)), _full_spec((1, HID // 2)),
                  _full_spec((NODE_DIM, HID)), _full_spec((HID, HID)),
                  _full_spec((1, HID)), _full_spec((1, HID)),
                  _full_spec((1, HID)), _full_spec((1, 1)),
                  _full_spec((HID, 1)), _full_spec((1, 1))],
        out_specs=_skinny_spec(),
        out_shape=jax.ShapeDtypeStruct((128, 128), jnp.float32),
    )(val, ts, predm, padm, h1, g, cnta, cntb, e, idx2, wenc, benc, ptok,
      ifreq, wn, wo, bo, gam, bet, lam, wp, bp)


def kernel(batch_value, batch_timestamp, batch_var_idx, batch_pred_mask,
           batch_pad_mask, W_enc, b_enc, pred_token, nodevec_0, W_node_0,
           W_msg_0, W_out_0, b_out_0, gamma_0, beta_0, nodevec_1, W_node_1,
           W_msg_1, W_out_1, b_out_1, gamma_1, beta_1, lambda_, W_proj,
           b_proj):
    val = batch_value.reshape(128, 128)
    ts = batch_timestamp.reshape(128, 128)
    predm = batch_pred_mask.reshape(128, 128)
    padm = batch_pad_mask.reshape(128, 128)
    idx = batch_var_idx.reshape(NT).astype(jnp.int32)
    idx2 = idx.reshape(128, 128)
    nv0 = nodevec_0.reshape(NUM_NODES // 2, 2 * NODE_DIM)
    nv1 = nodevec_1.reshape(NUM_NODES // 2, 2 * NODE_DIM)
    pad_flat = batch_pad_mask.reshape(NT)
    wenc = W_enc.reshape(1, HID)
    benc = b_enc.reshape(1, HID)
    ptok = pred_token.reshape(1, HID)
    half = HID // 2
    ifreq = (1.0 / (10000.0 ** (jnp.arange(half, dtype=jnp.float32) / half))
             ).reshape(1, half)
    bo0 = b_out_0.reshape(1, HID)
    bo1 = b_out_1.reshape(1, HID)
    gam0 = gamma_0.reshape(1, HID)
    bet0 = beta_0.reshape(1, HID)
    gam1 = gamma_1.reshape(1, HID)
    bet1 = beta_1.reshape(1, HID)
    lam = jnp.asarray(lambda_, jnp.float32).reshape(1, 1)
    bp = b_proj.reshape(1, 1)
    zeros = jnp.zeros((4 * CH, HID), jnp.float32)

    seg_ce = _make_seg(True, True, 5)
    seg_e = _make_seg(False, True, 5)
    seg = _make_seg(False, False, 5)

    m0 = _tc1(val, ts, predm, padm, wenc, benc, ptok, ifreq, W_msg_0)
    g0, cnt2, e0 = seg_ce(m0, idx, pad_flat, nv0, zeros)
    cnta = cnt2[0].reshape(128, 128)
    cntb = cnt2[1].reshape(128, 128)
    m1 = _tc2(g0, cnta, cntb, padm, W_msg_0)
    g1, e1 = seg_e(m1, idx, pad_flat, nv1, zeros)
    h1, m2 = _tc3(val, ts, predm, padm, g1, cnta, cntb, e0, idx2, wenc, benc,
                  ptok, ifreq, W_node_0, W_out_0, bo0, gam0, bet0, W_msg_1)
    g2 = seg(m2, idx, pad_flat, zeros)
    m3 = _tc2(g2, cnta, cntb, padm, W_msg_1)
    g3 = seg(m3, idx, pad_flat, zeros)
    out = _tc4(val, ts, predm, padm, h1, g3, cnta, cntb, e1, idx2, wenc,
               benc, ptok, ifreq, W_node_1, W_out_1, bo1, gam1, bet1, lam,
               W_proj, bp)
    return out.reshape(B, L, 1)
